# Initial kernel scaffold; baseline (speedup 1.0000x reference)
import jax
import jax.numpy as jnp
from jax.experimental import pallas as pl


def kernel(input):
    raise NotImplementedError("write your pallas kernel here")


# speedup vs baseline: 24.2154x; 24.2154x over previous
import jax
import jax.numpy as jnp
from jax.experimental import pallas as pl

_GAMMA = 2.0
_ITERS = 12
_ROWS_PER_BLOCK = 8


def _sparsemax_block(x_ref, o_ref):
    z = _GAMMA * x_ref[...]
    rowmax = jnp.max(z, axis=1, keepdims=True)
    tau0 = rowmax - 1.0

    def body(_, tau):
        mask = z > tau
        s = jnp.sum(jnp.where(mask, z, 0.0), axis=1, keepdims=True)
        c = jnp.sum(mask.astype(jnp.float32), axis=1, keepdims=True)
        return (s - 1.0) / c

    tau = jax.lax.fori_loop(0, _ITERS, body, tau0)
    o_ref[...] = jnp.maximum(z - tau, 0.0)


def kernel(input):
    bs, dim = input.shape
    r = _ROWS_PER_BLOCK
    return pl.pallas_call(
        _sparsemax_block,
        grid=(bs // r,),
        in_specs=[pl.BlockSpec((r, dim), lambda i: (i, 0))],
        out_specs=pl.BlockSpec((r, dim), lambda i: (i, 0)),
        out_shape=jax.ShapeDtypeStruct((bs, dim), input.dtype),
    )(input)


# speedup vs baseline: 34.4025x; 1.4207x over previous
import jax
import jax.numpy as jnp
from jax.experimental import pallas as pl

_GAMMA = 2.0
_ITERS = 8
_ROWS_PER_BLOCK = 8


def _sparsemax_block(x_ref, o_ref):
    z = _GAMMA * x_ref[...]
    rowmax = jnp.max(z, axis=1, keepdims=True)
    tau0 = rowmax - 1.0

    def body(_, tau):
        mask = z > tau
        s = jnp.sum(jnp.where(mask, z, 0.0), axis=1, keepdims=True)
        c = jnp.sum(mask.astype(jnp.float32), axis=1, keepdims=True)
        return (s - 1.0) / c

    tau = jax.lax.fori_loop(0, _ITERS, body, tau0)
    o_ref[...] = jnp.maximum(z - tau, 0.0)


def kernel(input):
    bs, dim = input.shape
    r = _ROWS_PER_BLOCK
    return pl.pallas_call(
        _sparsemax_block,
        grid=(bs // r,),
        in_specs=[pl.BlockSpec((r, dim), lambda i: (i, 0))],
        out_specs=pl.BlockSpec((r, dim), lambda i: (i, 0)),
        out_shape=jax.ShapeDtypeStruct((bs, dim), input.dtype),
    )(input)


# speedup vs baseline: 40.1067x; 1.1658x over previous
import jax
import jax.numpy as jnp
from jax.experimental import pallas as pl

_GAMMA = 2.0
_ITERS = 8
_ROWS_PER_BLOCK = 8


def _sparsemax_block(x_ref, o_ref):
    x0 = x_ref[...]
    rowmax = jnp.max(x0, axis=1, keepdims=True)
    t0 = rowmax - (1.0 / _GAMMA)

    def body(_, t):
        x = x_ref[...]
        mask = x > t
        s = jnp.sum(jnp.where(mask, x, 0.0), axis=1, keepdims=True)
        c = jnp.sum(mask.astype(jnp.float32), axis=1, keepdims=True)
        return (s - (1.0 / _GAMMA)) / c

    t = jax.lax.fori_loop(0, _ITERS, body, t0)
    o_ref[...] = _GAMMA * jnp.maximum(x_ref[...] - t, 0.0)


def kernel(input):
    bs, dim = input.shape
    r = _ROWS_PER_BLOCK
    return pl.pallas_call(
        _sparsemax_block,
        grid=(bs // r,),
        in_specs=[pl.BlockSpec((r, dim), lambda i: (i, 0))],
        out_specs=pl.BlockSpec((r, dim), lambda i: (i, 0)),
        out_shape=jax.ShapeDtypeStruct((bs, dim), input.dtype),
    )(input)


# speedup vs baseline: 54.8356x; 1.3672x over previous
import jax
import jax.numpy as jnp
from jax.experimental import pallas as pl

_GAMMA = 2.0
_ITERS = 8
_ROWS_PER_BLOCK = 32


def _sparsemax_block(x_ref, o_ref):
    x0 = x_ref[...]
    rowmax = jnp.max(x0, axis=1, keepdims=True)
    t0 = rowmax - (1.0 / _GAMMA)

    def body(_, t):
        x = x_ref[...]
        mask = x > t
        s = jnp.sum(jnp.where(mask, x, 0.0), axis=1, keepdims=True)
        c = jnp.sum(mask.astype(jnp.float32), axis=1, keepdims=True)
        return (s - (1.0 / _GAMMA)) / c

    t = jax.lax.fori_loop(0, _ITERS, body, t0)
    o_ref[...] = _GAMMA * jnp.maximum(x_ref[...] - t, 0.0)


def kernel(input):
    bs, dim = input.shape
    r = _ROWS_PER_BLOCK
    return pl.pallas_call(
        _sparsemax_block,
        grid=(bs // r,),
        in_specs=[pl.BlockSpec((r, dim), lambda i: (i, 0))],
        out_specs=pl.BlockSpec((r, dim), lambda i: (i, 0)),
        out_shape=jax.ShapeDtypeStruct((bs, dim), input.dtype),
    )(input)


# speedup vs baseline: 56.8661x; 1.0370x over previous
import jax
import jax.numpy as jnp
from jax.experimental import pallas as pl

_GAMMA = 2.0
_ITERS = 8
_ROWS_PER_BLOCK = 64


def _sparsemax_block(x_ref, o_ref):
    x0 = x_ref[...]
    rowmax = jnp.max(x0, axis=1, keepdims=True)
    t0 = rowmax - (1.0 / _GAMMA)

    def body(_, t):
        x = x_ref[...]
        mask = x > t
        s = jnp.sum(jnp.where(mask, x, 0.0), axis=1, keepdims=True)
        c = jnp.sum(mask.astype(jnp.float32), axis=1, keepdims=True)
        return (s - (1.0 / _GAMMA)) / c

    t = jax.lax.fori_loop(0, _ITERS, body, t0)
    o_ref[...] = _GAMMA * jnp.maximum(x_ref[...] - t, 0.0)


def kernel(input):
    bs, dim = input.shape
    r = _ROWS_PER_BLOCK
    return pl.pallas_call(
        _sparsemax_block,
        grid=(bs // r,),
        in_specs=[pl.BlockSpec((r, dim), lambda i: (i, 0))],
        out_specs=pl.BlockSpec((r, dim), lambda i: (i, 0)),
        out_shape=jax.ShapeDtypeStruct((bs, dim), input.dtype),
    )(input)


# speedup vs baseline: 77.0804x; 1.3555x over previous
import jax
import jax.numpy as jnp
from jax.experimental import pallas as pl

_GAMMA = 2.0
_ITERS = 24
_ROWS_PER_BLOCK = 64


def _sparsemax_block(x_ref, o_ref):
    x0 = x_ref[...]
    rowmax = jnp.max(x0, axis=1, keepdims=True)
    t0 = rowmax - (1.0 / _GAMMA)

    def step(t):
        x = x_ref[...]
        mask = x > t
        s = jnp.sum(jnp.where(mask, x, 0.0), axis=1, keepdims=True)
        c = jnp.sum(mask.astype(jnp.float32), axis=1, keepdims=True)
        return (s - (1.0 / _GAMMA)) / c

    def cond(carry):
        k, t_prev, t = carry
        return jnp.logical_and(k < _ITERS, jnp.any(t_prev != t))

    def body(carry):
        k, _, t = carry
        return k + 1, t, step(t)

    _, _, t = jax.lax.while_loop(cond, body, (0, t0 - 1.0, t0))
    o_ref[...] = _GAMMA * jnp.maximum(x_ref[...] - t, 0.0)


def kernel(input):
    bs, dim = input.shape
    r = _ROWS_PER_BLOCK
    return pl.pallas_call(
        _sparsemax_block,
        grid=(bs // r,),
        in_specs=[pl.BlockSpec((r, dim), lambda i: (i, 0))],
        out_specs=pl.BlockSpec((r, dim), lambda i: (i, 0)),
        out_shape=jax.ShapeDtypeStruct((bs, dim), input.dtype),
    )(input)
